# Initial kernel scaffold; baseline (speedup 1.0000x reference)
import jax
import jax.numpy as jnp
from jax.experimental import pallas as pl


def kernel(x, edge_index, W1, b1, W2, b2, a, fc1_W, fc1_b, fc2_W, fc2_b):
    raise NotImplementedError("write your pallas kernel here")


# speedup vs baseline: 8.4965x; 8.4965x over previous
import functools

import jax
import jax.numpy as jnp
from jax import lax
from jax.experimental import pallas as pl
from jax.experimental.pallas import tpu as pltpu
from jax.experimental.pallas import tpu_sc as plsc

N = 10000
NP = 10240
E = 320000
D_IN = 128
D_HID = 256
D_OUT = 128

NC = 2
NS = 16
CH = 128
EP = 327680
RT1 = EP // NS // CH
RT2 = EP // (NC * NS) // CH
IB = 8
SLICE = NP // NS
BLK = 1024
GRID = NP // BLK

_f32 = jnp.float32


def _mesh():
    return plsc.VectorSubcoreMesh(core_axis_name="c", subcore_axis_name="s")


def _make_deg():
    @functools.partial(
        pl.kernel,
        out_type=jax.ShapeDtypeStruct((NC * NP, D_OUT), _f32),
        mesh=_mesh(),
        scratch_types=[
            pltpu.VMEM((IB, CH), jnp.int32),
            pltpu.VMEM((CH, D_OUT), _f32),
            pltpu.VMEM_SHARED((NP, D_OUT), _f32),
        ],
    )
    def deg_kernel(dst_hbm, ones_hbm, zeros_hbm, out_hbm, dst_v, ones_v, accum):
        c = lax.axis_index("c")
        s = lax.axis_index("s")
        w = c * NS + s
        pltpu.sync_copy(ones_hbm, ones_v)
        pltpu.sync_copy(zeros_hbm, accum.at[pl.ds(s * SLICE, SLICE)])
        plsc.subcore_barrier()

        def body(b, carry):
            pltpu.sync_copy(dst_hbm.at[w, pl.ds(b * IB, IB)], dst_v)
            for r in range(IB):
                pltpu.sync_copy(ones_v, accum.at[dst_v.at[r]], add=True)
            return carry

        lax.fori_loop(0, RT2 // IB, body, 0)
        plsc.subcore_barrier()
        pltpu.sync_copy(
            accum.at[pl.ds(s * SLICE, SLICE)],
            out_hbm.at[pl.ds(c * NP + s * SLICE, SLICE)],
        )

    return deg_kernel


def _make_agg(chunks, src_cstride, dst_cstride):
    @functools.partial(
        pl.kernel,
        out_type=jax.ShapeDtypeStruct((NC * NP, D_OUT), _f32),
        mesh=_mesh(),
        scratch_types=[
            pltpu.VMEM((IB, CH), jnp.int32),
            pltpu.VMEM((IB, CH), jnp.int32),
            pltpu.VMEM((CH, D_OUT), _f32),
            pltpu.VMEM_SHARED((NP, D_OUT), _f32),
        ],
    )
    def agg_kernel(table_hbm, src_hbm, dst_hbm, zeros_hbm, out_hbm,
                   src_v, dst_v, rows_v, accum):
        c = lax.axis_index("c")
        s = lax.axis_index("s")
        gs = c * src_cstride + s
        gd = c * dst_cstride + s
        pltpu.sync_copy(zeros_hbm, accum.at[pl.ds(s * SLICE, SLICE)])
        plsc.subcore_barrier()

        def body(b, carry):
            pltpu.sync_copy(src_hbm.at[gs, pl.ds(b * IB, IB)], src_v)
            pltpu.sync_copy(dst_hbm.at[gd, pl.ds(b * IB, IB)], dst_v)
            for r in range(IB):
                pltpu.sync_copy(table_hbm.at[src_v.at[r]], rows_v)
                pltpu.sync_copy(rows_v, accum.at[dst_v.at[r]], add=True)
            return carry

        lax.fori_loop(0, chunks // IB, body, 0)
        plsc.subcore_barrier()
        pltpu.sync_copy(
            accum.at[pl.ds(s * SLICE, SLICE)],
            out_hbm.at[pl.ds(c * NP + s * SLICE, SLICE)],
        )

    return agg_kernel


def _tc1_body(deg_ref, x_ref, w1_ref, h_ref, dinv_ref):
    deg = deg_ref[0, :, 0:1] + deg_ref[1, :, 0:1]
    d = lax.rsqrt(deg + 1.0)
    h = jnp.dot(x_ref[...], w1_ref[...], preferred_element_type=_f32)
    hs = h * d
    h_ref[0] = hs[:, :D_OUT]
    h_ref[1] = hs[:, D_OUT:]
    dinv_ref[...] = d


def _tc2_body(a_ref, agg_ref, h1_ref, dinv_ref, b1_ref, w2_ref, h2_ref):
    av = a_ref[0, 0]
    d = dinv_ref[...]
    t0 = d * (agg_ref[0] + h1_ref[0]) + b1_ref[:, :D_OUT]
    t1 = d * (agg_ref[1] + h1_ref[1]) + b1_ref[:, D_OUT:]
    p0 = jnp.where(t0 >= 0, t0, av * t0)
    p1 = jnp.where(t1 >= 0, t1, av * t1)
    h2 = jnp.dot(p0, w2_ref[:D_OUT, :], preferred_element_type=_f32)
    h2 = h2 + jnp.dot(p1, w2_ref[D_OUT:, :], preferred_element_type=_f32)
    h2_ref[...] = h2 * d


def _tc3_body(a_ref, aggp_ref, h2_ref, dinv_ref, b2_ref,
              f1w_ref, f1b_ref, f2w_ref, f2b_ref, out_ref, proj_ref):
    av = a_ref[0, 0]
    d = dinv_ref[...]
    g = d * (aggp_ref[0] + aggp_ref[1] + h2_ref[...]) + b2_ref[...]
    out = jnp.where(g >= 0, g, av * g)
    out_ref[...] = out
    pr = jnp.dot(out, f1w_ref[...], preferred_element_type=_f32) + f1b_ref[...]
    pr = jnp.maximum(pr, 0.0)
    proj_ref[...] = jnp.dot(pr, f2w_ref[...], preferred_element_type=_f32) + f2b_ref[...]


def _vspec(shape3=None, shape2=None, imap3=None, imap2=None):
    if shape3 is not None:
        return pl.BlockSpec(shape3, imap3 or (lambda i: (0, i, 0)))
    return pl.BlockSpec(shape2, imap2 or (lambda i: (i, 0)))


def _full2(shape):
    return pl.BlockSpec(shape, lambda i: (0, 0))


_SMEM_SPEC = pl.BlockSpec(memory_space=pltpu.SMEM)


def _tc1(degr3, x_pad, W1):
    return pl.pallas_call(
        _tc1_body,
        grid=(GRID,),
        in_specs=[
            pl.BlockSpec((2, BLK, D_OUT), lambda i: (0, i, 0)),
            pl.BlockSpec((BLK, D_IN), lambda i: (i, 0)),
            _full2((D_IN, D_HID)),
        ],
        out_specs=[
            pl.BlockSpec((2, BLK, D_OUT), lambda i: (0, i, 0)),
            pl.BlockSpec((BLK, 1), lambda i: (i, 0)),
        ],
        out_shape=[
            jax.ShapeDtypeStruct((NC, NP, D_OUT), _f32),
            jax.ShapeDtypeStruct((NP, 1), _f32),
        ],
    )(degr3, x_pad, W1)


def _tc2(a2d, agg3, h1_3, dinv, b1_2d, W2):
    return pl.pallas_call(
        _tc2_body,
        grid=(GRID,),
        in_specs=[
            _SMEM_SPEC,
            pl.BlockSpec((2, BLK, D_OUT), lambda i: (0, i, 0)),
            pl.BlockSpec((2, BLK, D_OUT), lambda i: (0, i, 0)),
            pl.BlockSpec((BLK, 1), lambda i: (i, 0)),
            _full2((1, D_HID)),
            _full2((D_HID, D_OUT)),
        ],
        out_specs=pl.BlockSpec((BLK, D_OUT), lambda i: (i, 0)),
        out_shape=jax.ShapeDtypeStruct((NP, D_OUT), _f32),
    )(a2d, agg3, h1_3, dinv, b1_2d, W2)


def _tc3(a2d, agg3, h2, dinv, b2_2d, fc1_W, f1b2, fc2_W, f2b2):
    return pl.pallas_call(
        _tc3_body,
        grid=(GRID,),
        in_specs=[
            _SMEM_SPEC,
            pl.BlockSpec((2, BLK, D_OUT), lambda i: (0, i, 0)),
            pl.BlockSpec((BLK, D_OUT), lambda i: (i, 0)),
            pl.BlockSpec((BLK, 1), lambda i: (i, 0)),
            _full2((1, D_OUT)),
            _full2((D_OUT, D_OUT)),
            _full2((1, D_OUT)),
            _full2((D_OUT, D_OUT)),
            _full2((1, D_OUT)),
        ],
        out_specs=[
            pl.BlockSpec((BLK, D_OUT), lambda i: (i, 0)),
            pl.BlockSpec((BLK, D_OUT), lambda i: (i, 0)),
        ],
        out_shape=[
            jax.ShapeDtypeStruct((NP, D_OUT), _f32),
            jax.ShapeDtypeStruct((NP, D_OUT), _f32),
        ],
    )(a2d, agg3, h2, dinv, b2_2d, fc1_W, f1b2, fc2_W, f2b2)


_deg_call = _make_deg()
_agg1_call = _make_agg(chunks=RT1, src_cstride=NS, dst_cstride=0)
_agg2_call = _make_agg(chunks=RT2, src_cstride=NS, dst_cstride=NS)


def kernel(x, edge_index, W1, b1, W2, b2, a, fc1_W, fc1_b, fc2_W, fc2_b):
    pad = jnp.full((EP - E,), N, jnp.int32)
    src = jnp.concatenate([edge_index[0].astype(jnp.int32), pad])
    dst = jnp.concatenate([edge_index[1].astype(jnp.int32), pad])
    src_l1 = jnp.concatenate([src, src + NP]).reshape(NC * NS, RT1, CH)
    dst_l1 = dst.reshape(NS, RT1, CH)
    src_l2 = src.reshape(NC * NS, RT2, CH)
    dst_l2 = dst.reshape(NC * NS, RT2, CH)

    x_pad = jnp.pad(x, ((0, NP - N), (0, 0)))
    zeros = jnp.zeros((SLICE, D_OUT), _f32)
    a2d = a.reshape(1, 1)
    b1_2d = b1.reshape(1, D_HID)
    b2_2d = b2.reshape(1, D_OUT)
    f1b2 = fc1_b.reshape(1, D_OUT)
    f2b2 = fc2_b.reshape(1, D_OUT)

    ones_r = jnp.ones((CH, D_OUT), _f32)
    degr = _deg_call(dst_l2, ones_r, zeros)
    h1_3, dinv = _tc1(degr.reshape(NC, NP, D_OUT), x_pad, W1)
    agg1 = _agg1_call(h1_3.reshape(NC * NP, D_OUT), src_l1, dst_l1, zeros)
    h2 = _tc2(a2d, agg1.reshape(NC, NP, D_OUT), h1_3, dinv, b1_2d, W2)
    agg2 = _agg2_call(h2, src_l2, dst_l2, zeros)
    out, proj = _tc3(a2d, agg2.reshape(NC, NP, D_OUT), h2, dinv, b2_2d,
                     fc1_W, f1b2, fc2_W, f2b2)
    return out[:N], proj[:N]


# speedup vs baseline: 9.5832x; 1.1279x over previous
import functools

import jax
import jax.numpy as jnp
from jax import lax
from jax.experimental import pallas as pl
from jax.experimental.pallas import tpu as pltpu
from jax.experimental.pallas import tpu_sc as plsc

N = 10000
NP = 10240
E = 320000
D_IN = 128
D_HID = 256
D_OUT = 128

NC = 2
NS = 16
CH = 128
EP = 327680
RT1 = EP // NS // CH
RT2 = EP // (NC * NS) // CH
IB = 8
SLICE = NP // NS
BLK = 1024
GRID = NP // BLK

_f32 = jnp.float32


def _mesh():
    return plsc.VectorSubcoreMesh(core_axis_name="c", subcore_axis_name="s")


def _make_deg():
    @functools.partial(
        pl.kernel,
        out_type=jax.ShapeDtypeStruct((NC * NP, D_OUT), _f32),
        mesh=_mesh(),
        scratch_types=[
            pltpu.VMEM((IB, CH), jnp.int32),
            pltpu.VMEM((CH, D_OUT), _f32),
            pltpu.VMEM_SHARED((NP, D_OUT), _f32),
        ],
    )
    def deg_kernel(dst_hbm, ones_hbm, zeros_hbm, out_hbm, dst_v, ones_v, accum):
        c = lax.axis_index("c")
        s = lax.axis_index("s")
        w = c * NS + s
        pltpu.sync_copy(ones_hbm, ones_v)
        pltpu.sync_copy(zeros_hbm, accum.at[pl.ds(s * SLICE, SLICE)])
        plsc.subcore_barrier()

        def body(b, carry):
            pltpu.sync_copy(dst_hbm.at[w, pl.ds(b * IB, IB)], dst_v)
            for r in range(IB):
                pltpu.sync_copy(ones_v, accum.at[dst_v.at[r]], add=True)
            return carry

        lax.fori_loop(0, RT2 // IB, body, 0)
        plsc.subcore_barrier()
        pltpu.sync_copy(
            accum.at[pl.ds(s * SLICE, SLICE)],
            out_hbm.at[pl.ds(c * NP + s * SLICE, SLICE)],
        )

    return deg_kernel


def _make_agg(chunks, src_cstride, dst_cstride):
    nb = chunks // IB

    @functools.partial(
        pl.kernel,
        out_type=jax.ShapeDtypeStruct((NC * NP, D_OUT), _f32),
        mesh=_mesh(),
        scratch_types=[
            pltpu.VMEM((IB, CH), jnp.int32),
            pltpu.VMEM((IB, CH), jnp.int32),
            pltpu.VMEM((IB, CH), jnp.int32),
            pltpu.VMEM((IB, CH), jnp.int32),
            pltpu.VMEM((CH, D_OUT), _f32),
            pltpu.VMEM((CH, D_OUT), _f32),
            pltpu.VMEM_SHARED((NP, D_OUT), _f32),
            pltpu.SemaphoreType.DMA,
            pltpu.SemaphoreType.DMA,
        ],
    )
    def agg_kernel(table_hbm, src_hbm, dst_hbm, zeros_hbm, out_hbm,
                   src0, src1, dst0, dst1, rows0, rows1, accum, sem0, sem1):
        c = lax.axis_index("c")
        s = lax.axis_index("s")
        gs = c * src_cstride + s
        gd = c * dst_cstride + s
        srcs = (src0, src1)
        dsts = (dst0, dst1)
        rows = (rows0, rows1)
        sems = (sem0, sem1)

        pltpu.sync_copy(zeros_hbm, accum.at[pl.ds(s * SLICE, SLICE)])
        plsc.subcore_barrier()

        def stage(b, sl):
            pltpu.sync_copy(src_hbm.at[gs, pl.ds(b * IB, IB)], srcs[sl])
            pltpu.sync_copy(dst_hbm.at[gd, pl.ds(b * IB, IB)], dsts[sl])

        def gissue(sl, r, p):
            pltpu.async_copy(table_hbm.at[srcs[sl].at[r]], rows[p], sems[p])

        def gwait(sl, r, p):
            pltpu.make_async_copy(table_hbm.at[srcs[sl].at[r]], rows[p],
                                  sems[p]).wait()

        stage(0, 0)
        gissue(0, 0, 0)

        def pair(b2, carry):
            for half in range(2):
                b = b2 * 2 + half
                sl = half
                nsl = 1 - half
                if half == 0:
                    stage(b + 1, nsl)
                else:
                    @pl.when(b2 < nb // 2 - 1)
                    def _():
                        stage(b + 1, nsl)
                for r in range(IB):
                    p = r % 2
                    gwait(sl, r, p)
                    if r < IB - 1:
                        gissue(sl, r + 1, 1 - p)
                    elif half == 0:
                        gissue(nsl, 0, 1 - p)
                    else:
                        @pl.when(b2 < nb // 2 - 1)
                        def _():
                            gissue(nsl, 0, 1 - p)
                    pltpu.sync_copy(rows[p], accum.at[dsts[sl].at[r]], add=True)
            return carry

        lax.fori_loop(0, nb // 2, pair, 0)
        plsc.subcore_barrier()
        pltpu.sync_copy(
            accum.at[pl.ds(s * SLICE, SLICE)],
            out_hbm.at[pl.ds(c * NP + s * SLICE, SLICE)],
        )

    return agg_kernel


def _tc1_body(deg_ref, x_ref, w1_ref, h_ref, dinv_ref):
    deg = deg_ref[0, :, 0:1] + deg_ref[1, :, 0:1]
    d = lax.rsqrt(deg + 1.0)
    h = jnp.dot(x_ref[...], w1_ref[...], preferred_element_type=_f32)
    hs = h * d
    h_ref[0] = hs[:, :D_OUT]
    h_ref[1] = hs[:, D_OUT:]
    dinv_ref[...] = d


def _tc2_body(a_ref, agg_ref, h1_ref, dinv_ref, b1_ref, w2_ref, h2_ref):
    av = a_ref[0, 0]
    d = dinv_ref[...]
    t0 = d * (agg_ref[0] + h1_ref[0]) + b1_ref[:, :D_OUT]
    t1 = d * (agg_ref[1] + h1_ref[1]) + b1_ref[:, D_OUT:]
    p0 = jnp.where(t0 >= 0, t0, av * t0)
    p1 = jnp.where(t1 >= 0, t1, av * t1)
    h2 = jnp.dot(p0, w2_ref[:D_OUT, :], preferred_element_type=_f32)
    h2 = h2 + jnp.dot(p1, w2_ref[D_OUT:, :], preferred_element_type=_f32)
    h2_ref[...] = h2 * d


def _tc3_body(a_ref, aggp_ref, h2_ref, dinv_ref, b2_ref,
              f1w_ref, f1b_ref, f2w_ref, f2b_ref, out_ref, proj_ref):
    av = a_ref[0, 0]
    d = dinv_ref[...]
    g = d * (aggp_ref[0] + aggp_ref[1] + h2_ref[...]) + b2_ref[...]
    out = jnp.where(g >= 0, g, av * g)
    out_ref[...] = out
    pr = jnp.dot(out, f1w_ref[...], preferred_element_type=_f32) + f1b_ref[...]
    pr = jnp.maximum(pr, 0.0)
    proj_ref[...] = jnp.dot(pr, f2w_ref[...], preferred_element_type=_f32) + f2b_ref[...]


def _vspec(shape3=None, shape2=None, imap3=None, imap2=None):
    if shape3 is not None:
        return pl.BlockSpec(shape3, imap3 or (lambda i: (0, i, 0)))
    return pl.BlockSpec(shape2, imap2 or (lambda i: (i, 0)))


def _full2(shape):
    return pl.BlockSpec(shape, lambda i: (0, 0))


_SMEM_SPEC = pl.BlockSpec(memory_space=pltpu.SMEM)


def _tc1(degr3, x_pad, W1):
    return pl.pallas_call(
        _tc1_body,
        grid=(GRID,),
        in_specs=[
            pl.BlockSpec((2, BLK, D_OUT), lambda i: (0, i, 0)),
            pl.BlockSpec((BLK, D_IN), lambda i: (i, 0)),
            _full2((D_IN, D_HID)),
        ],
        out_specs=[
            pl.BlockSpec((2, BLK, D_OUT), lambda i: (0, i, 0)),
            pl.BlockSpec((BLK, 1), lambda i: (i, 0)),
        ],
        out_shape=[
            jax.ShapeDtypeStruct((NC, NP, D_OUT), _f32),
            jax.ShapeDtypeStruct((NP, 1), _f32),
        ],
    )(degr3, x_pad, W1)


def _tc2(a2d, agg3, h1_3, dinv, b1_2d, W2):
    return pl.pallas_call(
        _tc2_body,
        grid=(GRID,),
        in_specs=[
            _SMEM_SPEC,
            pl.BlockSpec((2, BLK, D_OUT), lambda i: (0, i, 0)),
            pl.BlockSpec((2, BLK, D_OUT), lambda i: (0, i, 0)),
            pl.BlockSpec((BLK, 1), lambda i: (i, 0)),
            _full2((1, D_HID)),
            _full2((D_HID, D_OUT)),
        ],
        out_specs=pl.BlockSpec((BLK, D_OUT), lambda i: (i, 0)),
        out_shape=jax.ShapeDtypeStruct((NP, D_OUT), _f32),
    )(a2d, agg3, h1_3, dinv, b1_2d, W2)


def _tc3(a2d, agg3, h2, dinv, b2_2d, fc1_W, f1b2, fc2_W, f2b2):
    return pl.pallas_call(
        _tc3_body,
        grid=(GRID,),
        in_specs=[
            _SMEM_SPEC,
            pl.BlockSpec((2, BLK, D_OUT), lambda i: (0, i, 0)),
            pl.BlockSpec((BLK, D_OUT), lambda i: (i, 0)),
            pl.BlockSpec((BLK, 1), lambda i: (i, 0)),
            _full2((1, D_OUT)),
            _full2((D_OUT, D_OUT)),
            _full2((1, D_OUT)),
            _full2((D_OUT, D_OUT)),
            _full2((1, D_OUT)),
        ],
        out_specs=[
            pl.BlockSpec((BLK, D_OUT), lambda i: (i, 0)),
            pl.BlockSpec((BLK, D_OUT), lambda i: (i, 0)),
        ],
        out_shape=[
            jax.ShapeDtypeStruct((NP, D_OUT), _f32),
            jax.ShapeDtypeStruct((NP, D_OUT), _f32),
        ],
    )(a2d, agg3, h2, dinv, b2_2d, fc1_W, f1b2, fc2_W, f2b2)


_deg_call = _make_deg()
_agg1_call = _make_agg(chunks=RT1, src_cstride=NS, dst_cstride=0)
_agg2_call = _make_agg(chunks=RT2, src_cstride=NS, dst_cstride=NS)


def kernel(x, edge_index, W1, b1, W2, b2, a, fc1_W, fc1_b, fc2_W, fc2_b):
    pad = jnp.full((EP - E,), N, jnp.int32)
    src = jnp.concatenate([edge_index[0].astype(jnp.int32), pad])
    dst = jnp.concatenate([edge_index[1].astype(jnp.int32), pad])
    src_l1 = jnp.concatenate([src, src + NP]).reshape(NC * NS, RT1, CH)
    dst_l1 = dst.reshape(NS, RT1, CH)
    src_l2 = src.reshape(NC * NS, RT2, CH)
    dst_l2 = dst.reshape(NC * NS, RT2, CH)

    x_pad = jnp.pad(x, ((0, NP - N), (0, 0)))
    zeros = jnp.zeros((SLICE, D_OUT), _f32)
    a2d = a.reshape(1, 1)
    b1_2d = b1.reshape(1, D_HID)
    b2_2d = b2.reshape(1, D_OUT)
    f1b2 = fc1_b.reshape(1, D_OUT)
    f2b2 = fc2_b.reshape(1, D_OUT)

    ones_r = jnp.ones((CH, D_OUT), _f32)
    degr = _deg_call(dst_l2, ones_r, zeros)
    h1_3, dinv = _tc1(degr.reshape(NC, NP, D_OUT), x_pad, W1)
    agg1 = _agg1_call(h1_3.reshape(NC * NP, D_OUT), src_l1, dst_l1, zeros)
    h2 = _tc2(a2d, agg1.reshape(NC, NP, D_OUT), h1_3, dinv, b1_2d, W2)
    agg2 = _agg2_call(h2, src_l2, dst_l2, zeros)
    out, proj = _tc3(a2d, agg2.reshape(NC, NP, D_OUT), h2, dinv, b2_2d,
                     fc1_W, f1b2, fc2_W, f2b2)
    return out[:N], proj[:N]


# speedup vs baseline: 19.8593x; 2.0723x over previous
import functools

import jax
import jax.numpy as jnp
from jax import lax
from jax.experimental import pallas as pl
from jax.experimental.pallas import tpu as pltpu
from jax.experimental.pallas import tpu_sc as plsc

N = 10000
NP = 10240
E = 320000
D_IN = 128
D_HID = 256
D_OUT = 128

NC = 2
NS = 16
CH = 128
EP = 327680
RT1 = EP // NS // CH
RT2 = EP // (NC * NS) // CH
IB = 8
SLICE = NP // NS
BLK = 1024
GRID = NP // BLK

_f32 = jnp.float32


def _mesh():
    return plsc.VectorSubcoreMesh(core_axis_name="c", subcore_axis_name="s")


def _make_deg():
    @functools.partial(
        pl.kernel,
        out_type=jax.ShapeDtypeStruct((NC * NP, D_OUT), _f32),
        mesh=_mesh(),
        scratch_types=[
            pltpu.VMEM((IB, CH), jnp.int32),
            pltpu.VMEM((CH, D_OUT), _f32),
            pltpu.VMEM_SHARED((NP, D_OUT), _f32),
        ],
    )
    def deg_kernel(dst_hbm, ones_hbm, zeros_hbm, out_hbm, dst_v, ones_v, accum):
        c = lax.axis_index("c")
        s = lax.axis_index("s")
        w = c * NS + s
        pltpu.sync_copy(ones_hbm, ones_v)
        pltpu.sync_copy(zeros_hbm, accum.at[pl.ds(s * SLICE, SLICE)])
        plsc.subcore_barrier()

        def body(b, carry):
            pltpu.sync_copy(dst_hbm.at[w, pl.ds(b * IB, IB)], dst_v)
            for r in range(IB):
                pltpu.sync_copy(ones_v, accum.at[dst_v.at[r]], add=True)
            return carry

        lax.fori_loop(0, RT2 // IB, body, 0)
        plsc.subcore_barrier()
        pltpu.sync_copy(
            accum.at[pl.ds(s * SLICE, SLICE)],
            out_hbm.at[pl.ds(c * NP + s * SLICE, SLICE)],
        )

    return deg_kernel


def _make_agg(chunks, src_cstride, dst_cstride):
    nb = chunks // IB

    @functools.partial(
        pl.kernel,
        out_type=jax.ShapeDtypeStruct((NC * NP, D_OUT), _f32),
        mesh=_mesh(),
        scratch_types=[
            pltpu.VMEM((IB, CH), jnp.int32),
            pltpu.VMEM((IB, CH), jnp.int32),
            pltpu.VMEM((IB, CH), jnp.int32),
            pltpu.VMEM((IB, CH), jnp.int32),
            pltpu.VMEM((CH, D_OUT), _f32),
            pltpu.VMEM((CH, D_OUT), _f32),
            pltpu.VMEM_SHARED((NP, D_OUT), _f32),
            pltpu.SemaphoreType.DMA,
            pltpu.SemaphoreType.DMA,
        ],
    )
    def agg_kernel(table_hbm, src_hbm, dst_hbm, zeros_hbm, out_hbm,
                   src0, src1, dst0, dst1, rows0, rows1, accum, sem0, sem1):
        c = lax.axis_index("c")
        s = lax.axis_index("s")
        gs = c * src_cstride + s
        gd = c * dst_cstride + s
        srcs = (src0, src1)
        dsts = (dst0, dst1)
        rows = (rows0, rows1)
        sems = (sem0, sem1)

        pltpu.sync_copy(zeros_hbm, accum.at[pl.ds(s * SLICE, SLICE)])
        plsc.subcore_barrier()

        def stage(b, sl):
            pltpu.sync_copy(src_hbm.at[gs, pl.ds(b * IB, IB)], srcs[sl])
            pltpu.sync_copy(dst_hbm.at[gd, pl.ds(b * IB, IB)], dsts[sl])

        def gissue(sl, r, p):
            pltpu.async_copy(table_hbm.at[srcs[sl].at[r]], rows[p], sems[p])

        def gwait(sl, r, p):
            pltpu.make_async_copy(table_hbm.at[srcs[sl].at[r]], rows[p],
                                  sems[p]).wait()

        stage(0, 0)
        gissue(0, 0, 0)

        def pair(b2, carry):
            for half in range(2):
                b = b2 * 2 + half
                sl = half
                nsl = 1 - half
                if half == 0:
                    stage(b + 1, nsl)
                else:
                    @pl.when(b2 < nb // 2 - 1)
                    def _():
                        stage(b + 1, nsl)
                for r in range(IB):
                    p = r % 2
                    gwait(sl, r, p)
                    if r < IB - 1:
                        gissue(sl, r + 1, 1 - p)
                    elif half == 0:
                        gissue(nsl, 0, 1 - p)
                    else:
                        @pl.when(b2 < nb // 2 - 1)
                        def _():
                            gissue(nsl, 0, 1 - p)
                    pltpu.sync_copy(rows[p], accum.at[dsts[sl].at[r]], add=True)
            return carry

        lax.fori_loop(0, nb // 2, pair, 0)
        plsc.subcore_barrier()
        pltpu.sync_copy(
            accum.at[pl.ds(s * SLICE, SLICE)],
            out_hbm.at[pl.ds(c * NP + s * SLICE, SLICE)],
        )

    return agg_kernel


def _tc1_body(deg_ref, x_ref, w1_ref, h_ref, dinv_ref):
    deg = deg_ref[0, :, 0:1] + deg_ref[1, :, 0:1]
    d = lax.rsqrt(deg + 1.0)
    h = jnp.dot(x_ref[...], w1_ref[...], preferred_element_type=_f32)
    hs = h * d
    h_ref[0] = hs[:, :D_OUT]
    h_ref[1] = hs[:, D_OUT:]
    dinv_ref[...] = d


def _tc2_body(a_ref, agg_ref, h1_ref, dinv_ref, b1_ref, w2_ref, h2_ref):
    av = a_ref[0, 0]
    d = dinv_ref[...]
    t0 = d * (agg_ref[0] + h1_ref[0]) + b1_ref[:, :D_OUT]
    t1 = d * (agg_ref[1] + h1_ref[1]) + b1_ref[:, D_OUT:]
    p0 = jnp.where(t0 >= 0, t0, av * t0)
    p1 = jnp.where(t1 >= 0, t1, av * t1)
    h2 = jnp.dot(p0, w2_ref[:D_OUT, :], preferred_element_type=_f32)
    h2 = h2 + jnp.dot(p1, w2_ref[D_OUT:, :], preferred_element_type=_f32)
    h2_ref[...] = h2 * d


def _tc3_body(a_ref, aggp_ref, h2_ref, dinv_ref, b2_ref,
              f1w_ref, f1b_ref, f2w_ref, f2b_ref, out_ref, proj_ref):
    av = a_ref[0, 0]
    d = dinv_ref[...]
    g = d * (aggp_ref[0] + aggp_ref[1] + h2_ref[...]) + b2_ref[...]
    out = jnp.where(g >= 0, g, av * g)
    out_ref[...] = out
    pr = jnp.dot(out, f1w_ref[...], preferred_element_type=_f32) + f1b_ref[...]
    pr = jnp.maximum(pr, 0.0)
    proj_ref[...] = jnp.dot(pr, f2w_ref[...], preferred_element_type=_f32) + f2b_ref[...]


def _vspec(shape3=None, shape2=None, imap3=None, imap2=None):
    if shape3 is not None:
        return pl.BlockSpec(shape3, imap3 or (lambda i: (0, i, 0)))
    return pl.BlockSpec(shape2, imap2 or (lambda i: (i, 0)))


def _full2(shape):
    return pl.BlockSpec(shape, lambda i: (0, 0))


_SMEM_SPEC = pl.BlockSpec(memory_space=pltpu.SMEM)


def _tc1(degr3, x_pad, W1):
    return pl.pallas_call(
        _tc1_body,
        grid=(GRID,),
        in_specs=[
            pl.BlockSpec((2, BLK, D_OUT), lambda i: (0, i, 0)),
            pl.BlockSpec((BLK, D_IN), lambda i: (i, 0)),
            _full2((D_IN, D_HID)),
        ],
        out_specs=[
            pl.BlockSpec((2, BLK, D_OUT), lambda i: (0, i, 0)),
            pl.BlockSpec((BLK, 1), lambda i: (i, 0)),
        ],
        out_shape=[
            jax.ShapeDtypeStruct((NC, NP, D_OUT), _f32),
            jax.ShapeDtypeStruct((NP, 1), _f32),
        ],
    )(degr3, x_pad, W1)


def _tc2(a2d, agg3, h1_3, dinv, b1_2d, W2):
    return pl.pallas_call(
        _tc2_body,
        grid=(GRID,),
        in_specs=[
            _SMEM_SPEC,
            pl.BlockSpec((2, BLK, D_OUT), lambda i: (0, i, 0)),
            pl.BlockSpec((2, BLK, D_OUT), lambda i: (0, i, 0)),
            pl.BlockSpec((BLK, 1), lambda i: (i, 0)),
            _full2((1, D_HID)),
            _full2((D_HID, D_OUT)),
        ],
        out_specs=pl.BlockSpec((BLK, D_OUT), lambda i: (i, 0)),
        out_shape=jax.ShapeDtypeStruct((NP, D_OUT), _f32),
    )(a2d, agg3, h1_3, dinv, b1_2d, W2)


def _tc3(a2d, agg3, h2, dinv, b2_2d, fc1_W, f1b2, fc2_W, f2b2):
    return pl.pallas_call(
        _tc3_body,
        grid=(GRID,),
        in_specs=[
            _SMEM_SPEC,
            pl.BlockSpec((2, BLK, D_OUT), lambda i: (0, i, 0)),
            pl.BlockSpec((BLK, D_OUT), lambda i: (i, 0)),
            pl.BlockSpec((BLK, 1), lambda i: (i, 0)),
            _full2((1, D_OUT)),
            _full2((D_OUT, D_OUT)),
            _full2((1, D_OUT)),
            _full2((D_OUT, D_OUT)),
            _full2((1, D_OUT)),
        ],
        out_specs=[
            pl.BlockSpec((BLK, D_OUT), lambda i: (i, 0)),
            pl.BlockSpec((BLK, D_OUT), lambda i: (i, 0)),
        ],
        out_shape=[
            jax.ShapeDtypeStruct((NP, D_OUT), _f32),
            jax.ShapeDtypeStruct((NP, D_OUT), _f32),
        ],
    )(a2d, agg3, h2, dinv, b2_2d, fc1_W, f1b2, fc2_W, f2b2)


_deg_call = _make_deg()
_agg1_call = _make_agg(chunks=RT1, src_cstride=NS, dst_cstride=0)
_agg2_call = _make_agg(chunks=RT2, src_cstride=NS, dst_cstride=NS)


def kernel(x, edge_index, W1, b1, W2, b2, a, fc1_W, fc1_b, fc2_W, fc2_b):
    pad = (N + jnp.arange(EP - E, dtype=jnp.int32) % (NP - N)).astype(jnp.int32)
    src = jnp.concatenate([edge_index[0].astype(jnp.int32), pad])
    dst = jnp.concatenate([edge_index[1].astype(jnp.int32), pad])
    src_l1 = jnp.concatenate([src, src + NP]).reshape(NC * NS, RT1, CH)
    dst_l1 = dst.reshape(NS, RT1, CH)
    src_l2 = src.reshape(NC * NS, RT2, CH)
    dst_l2 = dst.reshape(NC * NS, RT2, CH)

    x_pad = jnp.pad(x, ((0, NP - N), (0, 0)))
    zeros = jnp.zeros((SLICE, D_OUT), _f32)
    a2d = a.reshape(1, 1)
    b1_2d = b1.reshape(1, D_HID)
    b2_2d = b2.reshape(1, D_OUT)
    f1b2 = fc1_b.reshape(1, D_OUT)
    f2b2 = fc2_b.reshape(1, D_OUT)

    ones_r = jnp.ones((CH, D_OUT), _f32)
    degr = _deg_call(dst_l2, ones_r, zeros)
    h1_3, dinv = _tc1(degr.reshape(NC, NP, D_OUT), x_pad, W1)
    agg1 = _agg1_call(h1_3.reshape(NC * NP, D_OUT), src_l1, dst_l1, zeros)
    h2 = _tc2(a2d, agg1.reshape(NC, NP, D_OUT), h1_3, dinv, b1_2d, W2)
    agg2 = _agg2_call(h2, src_l2, dst_l2, zeros)
    out, proj = _tc3(a2d, agg2.reshape(NC, NP, D_OUT), h2, dinv, b2_2d,
                     fc1_W, f1b2, fc2_W, f2b2)
    return out[:N], proj[:N]


# speedup vs baseline: 20.4360x; 1.0290x over previous
import functools

import jax
import jax.numpy as jnp
from jax import lax
from jax.experimental import pallas as pl
from jax.experimental.pallas import tpu as pltpu
from jax.experimental.pallas import tpu_sc as plsc

N = 10000
NP = 10240
E = 320000
D_IN = 128
D_HID = 256
D_OUT = 128

NC = 2
NS = 16
CH = 128
EP = 327680
RT1 = EP // NS // CH
RT2 = EP // (NC * NS) // CH
IB = 8
SLICE = NP // NS
BLK = 1024
GRID = NP // BLK

_f32 = jnp.float32


def _mesh():
    return plsc.VectorSubcoreMesh(core_axis_name="c", subcore_axis_name="s")


def _make_deg():
    @functools.partial(
        pl.kernel,
        out_type=jax.ShapeDtypeStruct((NC * NP, D_OUT), _f32),
        mesh=_mesh(),
        scratch_types=[
            pltpu.VMEM((IB, CH), jnp.int32),
            pltpu.VMEM((CH, D_OUT), _f32),
            pltpu.VMEM_SHARED((NP, D_OUT), _f32),
        ],
    )
    def deg_kernel(dst_hbm, ones_hbm, zeros_hbm, out_hbm, dst_v, ones_v, accum):
        c = lax.axis_index("c")
        s = lax.axis_index("s")
        w = c * NS + s
        pltpu.sync_copy(ones_hbm, ones_v)
        pltpu.sync_copy(zeros_hbm, accum.at[pl.ds(s * SLICE, SLICE)])
        plsc.subcore_barrier()

        def body(b, carry):
            pltpu.sync_copy(dst_hbm.at[w, pl.ds(b * IB, IB)], dst_v)
            for r in range(IB):
                pltpu.sync_copy(ones_v, accum.at[dst_v.at[r]], add=True)
            return carry

        lax.fori_loop(0, RT2 // IB, body, 0)
        plsc.subcore_barrier()
        pltpu.sync_copy(
            accum.at[pl.ds(s * SLICE, SLICE)],
            out_hbm.at[pl.ds(c * NP + s * SLICE, SLICE)],
        )

    return deg_kernel


def _make_agg(chunks, src_cstride, dst_cstride):
    nb = chunks // IB

    @functools.partial(
        pl.kernel,
        out_type=jax.ShapeDtypeStruct((NC * NP, D_OUT), _f32),
        mesh=_mesh(),
        scratch_types=[
            pltpu.VMEM((IB, CH), jnp.int32),
            pltpu.VMEM((IB, CH), jnp.int32),
            pltpu.VMEM((IB, CH), jnp.int32),
            pltpu.VMEM((IB, CH), jnp.int32),
            pltpu.VMEM((CH, D_OUT), _f32),
            pltpu.VMEM((CH, D_OUT), _f32),
            pltpu.VMEM_SHARED((NP, D_OUT), _f32),
            pltpu.SemaphoreType.DMA,
            pltpu.SemaphoreType.DMA,
            pltpu.SemaphoreType.DMA,
            pltpu.SemaphoreType.DMA,
        ],
    )
    def agg_kernel(table_hbm, src_hbm, dst_hbm, zeros_hbm, out_hbm,
                   src0, src1, dst0, dst1, rows0, rows1, accum,
                   sem0, sem1, semi0, semi1):
        c = lax.axis_index("c")
        s = lax.axis_index("s")
        gs = c * src_cstride + s
        gd = c * dst_cstride + s
        srcs = (src0, src1)
        dsts = (dst0, dst1)
        rows = (rows0, rows1)
        sems = (sem0, sem1)
        semis = (semi0, semi1)

        pltpu.sync_copy(zeros_hbm, accum.at[pl.ds(s * SLICE, SLICE)])
        plsc.subcore_barrier()

        def stage(b, sl):
            pltpu.async_copy(src_hbm.at[gs, pl.ds(b * IB, IB)], srcs[sl], semis[sl])
            pltpu.async_copy(dst_hbm.at[gd, pl.ds(b * IB, IB)], dsts[sl], semis[sl])

        def stage_wait(b, sl):
            pltpu.make_async_copy(src_hbm.at[gs, pl.ds(b * IB, IB)], srcs[sl],
                                  semis[sl]).wait()
            pltpu.make_async_copy(dst_hbm.at[gd, pl.ds(b * IB, IB)], dsts[sl],
                                  semis[sl]).wait()

        def gissue(sl, r, p):
            pltpu.async_copy(table_hbm.at[srcs[sl].at[r]], rows[p], sems[p])

        def gwait(sl, r, p):
            pltpu.make_async_copy(table_hbm.at[srcs[sl].at[r]], rows[p],
                                  sems[p]).wait()

        stage(0, 0)
        stage_wait(0, 0)
        gissue(0, 0, 0)

        def pair(b2, carry):
            for half in range(2):
                b = b2 * 2 + half
                sl = half
                nsl = 1 - half
                if half == 0:
                    stage(b + 1, nsl)
                else:
                    @pl.when(b2 < nb // 2 - 1)
                    def _():
                        stage(b + 1, nsl)
                for r in range(IB):
                    p = r % 2
                    gwait(sl, r, p)
                    if r < IB - 1:
                        gissue(sl, r + 1, 1 - p)
                    elif half == 0:
                        stage_wait(b + 1, nsl)
                        gissue(nsl, 0, 1 - p)
                    else:
                        @pl.when(b2 < nb // 2 - 1)
                        def _():
                            stage_wait(b + 1, nsl)
                            gissue(nsl, 0, 1 - p)
                    pltpu.sync_copy(rows[p], accum.at[dsts[sl].at[r]], add=True)
            return carry

        lax.fori_loop(0, nb // 2, pair, 0)
        plsc.subcore_barrier()
        pltpu.sync_copy(
            accum.at[pl.ds(s * SLICE, SLICE)],
            out_hbm.at[pl.ds(c * NP + s * SLICE, SLICE)],
        )

    return agg_kernel


def _tc1_body(deg_ref, x_ref, w1_ref, h_ref, dinv_ref):
    deg = deg_ref[0, :, 0:1] + deg_ref[1, :, 0:1]
    d = lax.rsqrt(deg + 1.0)
    h = jnp.dot(x_ref[...], w1_ref[...], preferred_element_type=_f32)
    hs = h * d
    h_ref[0] = hs[:, :D_OUT]
    h_ref[1] = hs[:, D_OUT:]
    dinv_ref[...] = d


def _tc2_body(a_ref, agg_ref, h1_ref, dinv_ref, b1_ref, w2_ref, h2_ref):
    av = a_ref[0, 0]
    d = dinv_ref[...]
    t0 = d * (agg_ref[0] + h1_ref[0]) + b1_ref[:, :D_OUT]
    t1 = d * (agg_ref[1] + h1_ref[1]) + b1_ref[:, D_OUT:]
    p0 = jnp.where(t0 >= 0, t0, av * t0)
    p1 = jnp.where(t1 >= 0, t1, av * t1)
    h2 = jnp.dot(p0, w2_ref[:D_OUT, :], preferred_element_type=_f32)
    h2 = h2 + jnp.dot(p1, w2_ref[D_OUT:, :], preferred_element_type=_f32)
    h2_ref[...] = h2 * d


def _tc3_body(a_ref, aggp_ref, h2_ref, dinv_ref, b2_ref,
              f1w_ref, f1b_ref, f2w_ref, f2b_ref, out_ref, proj_ref):
    av = a_ref[0, 0]
    d = dinv_ref[...]
    g = d * (aggp_ref[0] + aggp_ref[1] + h2_ref[...]) + b2_ref[...]
    out = jnp.where(g >= 0, g, av * g)
    out_ref[...] = out
    pr = jnp.dot(out, f1w_ref[...], preferred_element_type=_f32) + f1b_ref[...]
    pr = jnp.maximum(pr, 0.0)
    proj_ref[...] = jnp.dot(pr, f2w_ref[...], preferred_element_type=_f32) + f2b_ref[...]


def _vspec(shape3=None, shape2=None, imap3=None, imap2=None):
    if shape3 is not None:
        return pl.BlockSpec(shape3, imap3 or (lambda i: (0, i, 0)))
    return pl.BlockSpec(shape2, imap2 or (lambda i: (i, 0)))


def _full2(shape):
    return pl.BlockSpec(shape, lambda i: (0, 0))


_SMEM_SPEC = pl.BlockSpec(memory_space=pltpu.SMEM)


def _tc1(degr3, x_pad, W1):
    return pl.pallas_call(
        _tc1_body,
        grid=(GRID,),
        in_specs=[
            pl.BlockSpec((2, BLK, D_OUT), lambda i: (0, i, 0)),
            pl.BlockSpec((BLK, D_IN), lambda i: (i, 0)),
            _full2((D_IN, D_HID)),
        ],
        out_specs=[
            pl.BlockSpec((2, BLK, D_OUT), lambda i: (0, i, 0)),
            pl.BlockSpec((BLK, 1), lambda i: (i, 0)),
        ],
        out_shape=[
            jax.ShapeDtypeStruct((NC, NP, D_OUT), _f32),
            jax.ShapeDtypeStruct((NP, 1), _f32),
        ],
    )(degr3, x_pad, W1)


def _tc2(a2d, agg3, h1_3, dinv, b1_2d, W2):
    return pl.pallas_call(
        _tc2_body,
        grid=(GRID,),
        in_specs=[
            _SMEM_SPEC,
            pl.BlockSpec((2, BLK, D_OUT), lambda i: (0, i, 0)),
            pl.BlockSpec((2, BLK, D_OUT), lambda i: (0, i, 0)),
            pl.BlockSpec((BLK, 1), lambda i: (i, 0)),
            _full2((1, D_HID)),
            _full2((D_HID, D_OUT)),
        ],
        out_specs=pl.BlockSpec((BLK, D_OUT), lambda i: (i, 0)),
        out_shape=jax.ShapeDtypeStruct((NP, D_OUT), _f32),
    )(a2d, agg3, h1_3, dinv, b1_2d, W2)


def _tc3(a2d, agg3, h2, dinv, b2_2d, fc1_W, f1b2, fc2_W, f2b2):
    return pl.pallas_call(
        _tc3_body,
        grid=(GRID,),
        in_specs=[
            _SMEM_SPEC,
            pl.BlockSpec((2, BLK, D_OUT), lambda i: (0, i, 0)),
            pl.BlockSpec((BLK, D_OUT), lambda i: (i, 0)),
            pl.BlockSpec((BLK, 1), lambda i: (i, 0)),
            _full2((1, D_OUT)),
            _full2((D_OUT, D_OUT)),
            _full2((1, D_OUT)),
            _full2((D_OUT, D_OUT)),
            _full2((1, D_OUT)),
        ],
        out_specs=[
            pl.BlockSpec((BLK, D_OUT), lambda i: (i, 0)),
            pl.BlockSpec((BLK, D_OUT), lambda i: (i, 0)),
        ],
        out_shape=[
            jax.ShapeDtypeStruct((NP, D_OUT), _f32),
            jax.ShapeDtypeStruct((NP, D_OUT), _f32),
        ],
    )(a2d, agg3, h2, dinv, b2_2d, fc1_W, f1b2, fc2_W, f2b2)


_deg_call = _make_deg()
_agg1_call = _make_agg(chunks=RT1, src_cstride=NS, dst_cstride=0)
_agg2_call = _make_agg(chunks=RT2, src_cstride=NS, dst_cstride=NS)


def kernel(x, edge_index, W1, b1, W2, b2, a, fc1_W, fc1_b, fc2_W, fc2_b):
    pad = (N + jnp.arange(EP - E, dtype=jnp.int32) % (NP - N)).astype(jnp.int32)
    src = jnp.concatenate([edge_index[0].astype(jnp.int32), pad])
    dst = jnp.concatenate([edge_index[1].astype(jnp.int32), pad])
    src_l1 = jnp.concatenate([src, src + NP]).reshape(NC * NS, RT1, CH)
    dst_l1 = dst.reshape(NS, RT1, CH)
    src_l2 = src.reshape(NC * NS, RT2, CH)
    dst_l2 = dst.reshape(NC * NS, RT2, CH)

    x_pad = jnp.pad(x, ((0, NP - N), (0, 0)))
    zeros = jnp.zeros((SLICE, D_OUT), _f32)
    a2d = a.reshape(1, 1)
    b1_2d = b1.reshape(1, D_HID)
    b2_2d = b2.reshape(1, D_OUT)
    f1b2 = fc1_b.reshape(1, D_OUT)
    f2b2 = fc2_b.reshape(1, D_OUT)

    ones_r = jnp.ones((CH, D_OUT), _f32)
    degr = _deg_call(dst_l2, ones_r, zeros)
    h1_3, dinv = _tc1(degr.reshape(NC, NP, D_OUT), x_pad, W1)
    agg1 = _agg1_call(h1_3.reshape(NC * NP, D_OUT), src_l1, dst_l1, zeros)
    h2 = _tc2(a2d, agg1.reshape(NC, NP, D_OUT), h1_3, dinv, b1_2d, W2)
    agg2 = _agg2_call(h2, src_l2, dst_l2, zeros)
    out, proj = _tc3(a2d, agg2.reshape(NC, NP, D_OUT), h2, dinv, b2_2d,
                     fc1_W, f1b2, fc2_W, f2b2)
    return out[:N], proj[:N]


# speedup vs baseline: 20.4841x; 1.0024x over previous
import functools

import jax
import jax.numpy as jnp
from jax import lax
from jax.experimental import pallas as pl
from jax.experimental.pallas import tpu as pltpu
from jax.experimental.pallas import tpu_sc as plsc

N = 10000
NP = 10240
E = 320000
D_IN = 128
D_HID = 256
D_OUT = 128

NC = 2
NS = 16
CH = 128
EP = 327680
RT1 = EP // NS // CH
RT2 = EP // (NC * NS) // CH
IB = 8
SLICE = NP // NS
BLK = 1024
GRID = NP // BLK
DW = 128

_f32 = jnp.float32


def _mesh():
    return plsc.VectorSubcoreMesh(core_axis_name="c", subcore_axis_name="s")


def _make_deg():
    @functools.partial(
        pl.kernel,
        out_type=jax.ShapeDtypeStruct((NC * NP, DW), _f32),
        mesh=_mesh(),
        scratch_types=[
            pltpu.VMEM((IB, CH), jnp.int32),
            pltpu.VMEM((CH, DW), _f32),
            pltpu.VMEM_SHARED((NP, DW), _f32),
        ],
    )
    def deg_kernel(dst_hbm, ones_hbm, zeros_hbm, out_hbm, dst_v, ones_v, accum):
        c = lax.axis_index("c")
        s = lax.axis_index("s")
        w = c * NS + s
        pltpu.sync_copy(ones_hbm, ones_v)
        pltpu.sync_copy(zeros_hbm, accum.at[pl.ds(s * SLICE, SLICE)])
        plsc.subcore_barrier()

        def body(b, carry):
            pltpu.sync_copy(dst_hbm.at[w, pl.ds(b * IB, IB)], dst_v)
            for r in range(IB):
                pltpu.sync_copy(ones_v, accum.at[dst_v.at[r]], add=True)
            return carry

        lax.fori_loop(0, RT2 // IB, body, 0)
        plsc.subcore_barrier()
        pltpu.sync_copy(
            accum.at[pl.ds(s * SLICE, SLICE)],
            out_hbm.at[pl.ds(c * NP + s * SLICE, SLICE)],
        )

    return deg_kernel


def _make_agg(chunks, src_cstride, dst_cstride):
    nb = chunks // IB

    @functools.partial(
        pl.kernel,
        out_type=jax.ShapeDtypeStruct((NC * NP, D_OUT), _f32),
        mesh=_mesh(),
        scratch_types=[
            pltpu.VMEM((IB, CH), jnp.int32),
            pltpu.VMEM((IB, CH), jnp.int32),
            pltpu.VMEM((IB, CH), jnp.int32),
            pltpu.VMEM((IB, CH), jnp.int32),
            pltpu.VMEM((CH, D_OUT), _f32),
            pltpu.VMEM((CH, D_OUT), _f32),
            pltpu.VMEM_SHARED((NP, D_OUT), _f32),
            pltpu.SemaphoreType.DMA,
            pltpu.SemaphoreType.DMA,
            pltpu.SemaphoreType.DMA,
            pltpu.SemaphoreType.DMA,
            pltpu.SemaphoreType.DMA,
            pltpu.SemaphoreType.DMA,
        ],
    )
    def agg_kernel(table_hbm, src_hbm, dst_hbm, zeros_hbm, out_hbm,
                   src0, src1, dst0, dst1, rows0, rows1, accum,
                   sem0, sem1, semi0, semi1, sems0, sems1):
        c = lax.axis_index("c")
        s = lax.axis_index("s")
        gs = c * src_cstride + s
        gd = c * dst_cstride + s
        srcs = (src0, src1)
        dsts = (dst0, dst1)
        rows = (rows0, rows1)
        sems = (sem0, sem1)
        semis = (semi0, semi1)
        semss = (sems0, sems1)

        pltpu.sync_copy(zeros_hbm, accum.at[pl.ds(s * SLICE, SLICE)])
        plsc.subcore_barrier()

        def stage(b, sl):
            pltpu.async_copy(src_hbm.at[gs, pl.ds(b * IB, IB)], srcs[sl], semis[sl])
            pltpu.async_copy(dst_hbm.at[gd, pl.ds(b * IB, IB)], dsts[sl], semis[sl])

        def stage_wait(b, sl):
            pltpu.make_async_copy(src_hbm.at[gs, pl.ds(b * IB, IB)], srcs[sl],
                                  semis[sl]).wait()
            pltpu.make_async_copy(dst_hbm.at[gd, pl.ds(b * IB, IB)], dsts[sl],
                                  semis[sl]).wait()

        def gissue(sl, r, p):
            pltpu.async_copy(table_hbm.at[srcs[sl].at[r]], rows[p], sems[p])

        def gwait(sl, r, p):
            pltpu.make_async_copy(table_hbm.at[srcs[sl].at[r]], rows[p],
                                  sems[p]).wait()

        def sissue(sl, r, p):
            pltpu.async_copy(rows[p], accum.at[dsts[sl].at[r]], semss[p],
                             add=True)

        def swait(sl, r, p):
            pltpu.make_async_copy(rows[p], accum.at[dsts[sl].at[r]],
                                  semss[p]).wait()

        stage(0, 0)
        stage_wait(0, 0)
        gissue(0, 0, 0)

        def pair(b2, carry):
            for half in range(2):
                b = b2 * 2 + half
                sl = half
                nsl = 1 - half
                if half == 0:
                    @pl.when(b2 > 0)
                    def _():
                        swait(nsl, IB - 1, 1)
                    stage(b + 1, nsl)
                else:
                    swait(nsl, IB - 1, 1)

                    @pl.when(b2 < nb // 2 - 1)
                    def _():
                        stage(b + 1, nsl)
                for r in range(IB):
                    p = r % 2
                    gwait(sl, r, p)
                    if r > 0:
                        swait(sl, r - 1, 1 - p)
                    if r < IB - 1:
                        gissue(sl, r + 1, 1 - p)
                    elif half == 0:
                        stage_wait(b + 1, nsl)
                        gissue(nsl, 0, 1 - p)
                    else:
                        @pl.when(b2 < nb // 2 - 1)
                        def _():
                            stage_wait(b + 1, nsl)
                            gissue(nsl, 0, 1 - p)
                    sissue(sl, r, p)
            return carry

        lax.fori_loop(0, nb // 2, pair, 0)
        swait(1, IB - 1, 1)
        plsc.subcore_barrier()
        pltpu.sync_copy(
            accum.at[pl.ds(s * SLICE, SLICE)],
            out_hbm.at[pl.ds(c * NP + s * SLICE, SLICE)],
        )

    return agg_kernel


def _tc1_body(deg_ref, x_ref, w1_ref, h_ref, dinv_ref):
    deg = deg_ref[0, :, 0:1] + deg_ref[1, :, 0:1]
    d = lax.rsqrt(deg + 1.0)
    h = jnp.dot(x_ref[...], w1_ref[...], preferred_element_type=_f32)
    hs = h * d
    h_ref[0] = hs[:, :D_OUT]
    h_ref[1] = hs[:, D_OUT:]
    dinv_ref[...] = d


def _tc2_body(a_ref, agg_ref, h1_ref, dinv_ref, b1_ref, w2_ref, h2_ref):
    av = a_ref[0, 0]
    d = dinv_ref[...]
    t0 = d * (agg_ref[0] + h1_ref[0]) + b1_ref[:, :D_OUT]
    t1 = d * (agg_ref[1] + h1_ref[1]) + b1_ref[:, D_OUT:]
    p0 = jnp.where(t0 >= 0, t0, av * t0)
    p1 = jnp.where(t1 >= 0, t1, av * t1)
    h2 = jnp.dot(p0, w2_ref[:D_OUT, :], preferred_element_type=_f32)
    h2 = h2 + jnp.dot(p1, w2_ref[D_OUT:, :], preferred_element_type=_f32)
    h2_ref[...] = h2 * d


def _tc3_body(a_ref, aggp_ref, h2_ref, dinv_ref, b2_ref,
              f1w_ref, f1b_ref, f2w_ref, f2b_ref, out_ref, proj_ref):
    av = a_ref[0, 0]
    d = dinv_ref[...]
    g = d * (aggp_ref[0] + aggp_ref[1] + h2_ref[...]) + b2_ref[...]
    out = jnp.where(g >= 0, g, av * g)
    out_ref[...] = out
    pr = jnp.dot(out, f1w_ref[...], preferred_element_type=_f32) + f1b_ref[...]
    pr = jnp.maximum(pr, 0.0)
    proj_ref[...] = jnp.dot(pr, f2w_ref[...], preferred_element_type=_f32) + f2b_ref[...]


def _vspec(shape3=None, shape2=None, imap3=None, imap2=None):
    if shape3 is not None:
        return pl.BlockSpec(shape3, imap3 or (lambda i: (0, i, 0)))
    return pl.BlockSpec(shape2, imap2 or (lambda i: (i, 0)))


def _full2(shape):
    return pl.BlockSpec(shape, lambda i: (0, 0))


_SMEM_SPEC = pl.BlockSpec(memory_space=pltpu.SMEM)


def _tc1(degr3, x_pad, W1):
    return pl.pallas_call(
        _tc1_body,
        grid=(GRID,),
        in_specs=[
            pl.BlockSpec((2, BLK, DW), lambda i: (0, i, 0)),
            pl.BlockSpec((BLK, D_IN), lambda i: (i, 0)),
            _full2((D_IN, D_HID)),
        ],
        out_specs=[
            pl.BlockSpec((2, BLK, D_OUT), lambda i: (0, i, 0)),
            pl.BlockSpec((BLK, 1), lambda i: (i, 0)),
        ],
        out_shape=[
            jax.ShapeDtypeStruct((NC, NP, D_OUT), _f32),
            jax.ShapeDtypeStruct((NP, 1), _f32),
        ],
    )(degr3, x_pad, W1)


def _tc2(a2d, agg3, h1_3, dinv, b1_2d, W2):
    return pl.pallas_call(
        _tc2_body,
        grid=(GRID,),
        in_specs=[
            _SMEM_SPEC,
            pl.BlockSpec((2, BLK, D_OUT), lambda i: (0, i, 0)),
            pl.BlockSpec((2, BLK, D_OUT), lambda i: (0, i, 0)),
            pl.BlockSpec((BLK, 1), lambda i: (i, 0)),
            _full2((1, D_HID)),
            _full2((D_HID, D_OUT)),
        ],
        out_specs=pl.BlockSpec((BLK, D_OUT), lambda i: (i, 0)),
        out_shape=jax.ShapeDtypeStruct((NP, D_OUT), _f32),
    )(a2d, agg3, h1_3, dinv, b1_2d, W2)


def _tc3(a2d, agg3, h2, dinv, b2_2d, fc1_W, f1b2, fc2_W, f2b2):
    return pl.pallas_call(
        _tc3_body,
        grid=(GRID,),
        in_specs=[
            _SMEM_SPEC,
            pl.BlockSpec((2, BLK, D_OUT), lambda i: (0, i, 0)),
            pl.BlockSpec((BLK, D_OUT), lambda i: (i, 0)),
            pl.BlockSpec((BLK, 1), lambda i: (i, 0)),
            _full2((1, D_OUT)),
            _full2((D_OUT, D_OUT)),
            _full2((1, D_OUT)),
            _full2((D_OUT, D_OUT)),
            _full2((1, D_OUT)),
        ],
        out_specs=[
            pl.BlockSpec((BLK, D_OUT), lambda i: (i, 0)),
            pl.BlockSpec((BLK, D_OUT), lambda i: (i, 0)),
        ],
        out_shape=[
            jax.ShapeDtypeStruct((NP, D_OUT), _f32),
            jax.ShapeDtypeStruct((NP, D_OUT), _f32),
        ],
    )(a2d, agg3, h2, dinv, b2_2d, fc1_W, f1b2, fc2_W, f2b2)


_deg_call = _make_deg()
_agg1_call = _make_agg(chunks=RT1, src_cstride=NS, dst_cstride=0)
_agg2_call = _make_agg(chunks=RT2, src_cstride=NS, dst_cstride=NS)


def kernel(x, edge_index, W1, b1, W2, b2, a, fc1_W, fc1_b, fc2_W, fc2_b):
    pad = (N + jnp.arange(EP - E, dtype=jnp.int32) % (NP - N)).astype(jnp.int32)
    src = jnp.concatenate([edge_index[0].astype(jnp.int32), pad])
    dst = jnp.concatenate([edge_index[1].astype(jnp.int32), pad])
    src_l1 = jnp.concatenate([src, src + NP]).reshape(NC * NS, RT1, CH)
    dst_l1 = dst.reshape(NS, RT1, CH)
    src_l2 = src.reshape(NC * NS, RT2, CH)
    dst_l2 = dst.reshape(NC * NS, RT2, CH)

    x_pad = jnp.pad(x, ((0, NP - N), (0, 0)))
    zeros = jnp.zeros((SLICE, D_OUT), _f32)
    a2d = a.reshape(1, 1)
    b1_2d = b1.reshape(1, D_HID)
    b2_2d = b2.reshape(1, D_OUT)
    f1b2 = fc1_b.reshape(1, D_OUT)
    f2b2 = fc2_b.reshape(1, D_OUT)

    ones_r = jnp.ones((CH, DW), _f32)
    zeros_d = jnp.zeros((SLICE, DW), _f32)
    degr = _deg_call(dst_l2, ones_r, zeros_d)
    h1_3, dinv = _tc1(degr.reshape(NC, NP, DW), x_pad, W1)
    agg1 = _agg1_call(h1_3.reshape(NC * NP, D_OUT), src_l1, dst_l1, zeros)
    h2 = _tc2(a2d, agg1.reshape(NC, NP, D_OUT), h1_3, dinv, b1_2d, W2)
    agg2 = _agg2_call(h2, src_l2, dst_l2, zeros)
    out, proj = _tc3(a2d, agg2.reshape(NC, NP, D_OUT), h2, dinv, b2_2d,
                     fc1_W, f1b2, fc2_W, f2b2)
    return out[:N], proj[:N]
